# Initial kernel scaffold; baseline (speedup 1.0000x reference)
import jax
import jax.numpy as jnp
from jax.experimental import pallas as pl


def kernel(features, edge_metapath_indices_0, edge_dst_0, edge_metapath_indices_1, edge_dst_1, gru_Wih_0, gru_Whh_0, gru_bih_0, gru_bhh_0, attn_0, gru_Wih_1, gru_Whh_1, gru_bih_1, gru_bhh_1, attn_1, fc1_w1, fc1_b1, fc1_w2, fc1_b2, fc1_w3, fc1_b3, fc2_w):
    raise NotImplementedError("write your pallas kernel here")


# speedup vs baseline: 15.7828x; 15.7828x over previous
import functools

import jax
import jax.numpy as jnp
from jax import lax
from jax.experimental import pallas as pl
from jax.experimental.pallas import tpu as pltpu
from jax.experimental.pallas import tpu_sc as plsc

N = 10000
E = 160000
L = 3
OUT = 64
NH = 2
H = NH * OUT
AV = 128
VW = 144

NC = 2
NS = 16
NW = NC * NS

E_PAD = 163840
CHUNK = 128
G_PER_W = 2 * L * E_PAD // NW
G_CHUNKS = G_PER_W // CHUNK
S_PER_T = E_PAD // NS
S_CHUNKS = S_PER_T // CHUNK
N_PER_T = N // NS

B2 = 640
B4 = 1000


def _make_gather():
    mesh = plsc.VectorSubcoreMesh(core_axis_name="c", subcore_axis_name="s")

    @functools.partial(
        pl.kernel,
        mesh=mesh,
        out_type=jax.ShapeDtypeStruct((2 * L * E_PAD, OUT), jnp.float32),
        compiler_params=pltpu.CompilerParams(use_tc_tiling_on_sc=False),
        scratch_types=[
            pltpu.VMEM((G_CHUNKS, CHUNK), jnp.int32),
            pltpu.VMEM((CHUNK, OUT), jnp.float32),
            pltpu.SemaphoreType.DMA,
        ],
    )
    def gather_k(feat_hbm, idx_hbm, out_hbm, idx_v, rows_v, sem):
        c = lax.axis_index("c")
        s = lax.axis_index("s")
        wid = s * NC + c
        pltpu.sync_copy(idx_hbm.at[wid], idx_v)

        def body(j, carry):
            pltpu.async_copy(feat_hbm.at[idx_v.at[j]], rows_v, sem).wait()
            pltpu.sync_copy(
                rows_v, out_hbm.at[pl.ds(wid * G_PER_W + j * CHUNK, CHUNK)])
            return carry

        lax.fori_loop(0, G_CHUNKS, body, 0)

    return gather_k


def _make_scatter():
    mesh = plsc.VectorSubcoreMesh(core_axis_name="c", subcore_axis_name="s")

    @functools.partial(
        pl.kernel,
        mesh=mesh,
        out_type=jax.ShapeDtypeStruct((2, N, VW), jnp.float32),
        compiler_params=pltpu.CompilerParams(use_tc_tiling_on_sc=False),
        scratch_types=[
            pltpu.VMEM((S_CHUNKS, CHUNK), jnp.int32),
            pltpu.VMEM((CHUNK, VW), jnp.float32),
            pltpu.VMEM_SHARED((N, VW), jnp.float32),
            pltpu.SemaphoreType.DMA,
        ],
    )
    def scatter_k(vals_hbm, dst_hbm, zeros_hbm, acc_hbm, idx_v, vbuf, shacc,
                  sem):
        c = lax.axis_index("c")
        s = lax.axis_index("s")
        pltpu.sync_copy(zeros_hbm, shacc.at[pl.ds(s * N_PER_T, N_PER_T)])
        pltpu.sync_copy(dst_hbm.at[c, s], idx_v)
        plsc.subcore_barrier()

        def body(j, carry):
            pltpu.sync_copy(
                vals_hbm.at[c, pl.ds(s * S_PER_T + j * CHUNK, CHUNK)], vbuf)
            pltpu.sync_copy(vbuf, shacc.at[idx_v.at[j]], add=True)
            return carry

        lax.fori_loop(0, S_CHUNKS, body, 0)
        plsc.subcore_barrier()
        pltpu.sync_copy(
            shacc.at[pl.ds(s * N_PER_T, N_PER_T)],
            acc_hbm.at[c, pl.ds(s * N_PER_T, N_PER_T)])

    return scatter_k


_gather_call = _make_gather()
_scatter_call = _make_scatter()


def _gru_attn_body(ed_ref, wih_ref, whh_ref, bih_ref, bhh_ref, attn_ref,
                   vals_ref):
    j = pl.program_id(1)
    wih = wih_ref[0]
    whh = whh_ref[0]
    bih = bih_ref[0]
    bhh = bhh_ref[0]
    att = attn_ref[0]

    gis = [
        jnp.dot(ed_ref[0, l], wih, preferred_element_type=jnp.float32) + bih
        for l in range(L)
    ]

    h = None
    for l in range(L):
        gi = gis[l]
        if h is None:
            gh = jnp.broadcast_to(bhh, gi.shape)
        else:
            gh = jnp.dot(h, whh, preferred_element_type=jnp.float32) + bhh
        r = jax.nn.sigmoid(gi[:, :H] + gh[:, :H])
        z = jax.nn.sigmoid(gi[:, H:2 * H] + gh[:, H:2 * H])
        n = jnp.tanh(gi[:, 2 * H:] + r * gh[:, 2 * H:])
        h = (1.0 - z) * n if l == 0 else (1.0 - z) * n + z * h

    a0 = jnp.sum(h[:, :OUT] * att[:, :OUT], axis=-1, keepdims=True)
    a1 = jnp.sum(h[:, OUT:] * att[:, OUT:], axis=-1, keepdims=True)
    a0 = jnp.where(a0 >= 0, a0, 0.01 * a0)
    a1 = jnp.where(a1 >= 0, a1, 0.01 * a1)
    p0 = jnp.exp(a0)
    p1 = jnp.exp(a1)

    e0 = j * B2 + lax.broadcasted_iota(jnp.int32, (B2, 1), 0)
    msk = (e0 < E).astype(jnp.float32)
    tail = jnp.concatenate(
        [p0, p1, jnp.zeros((B2, VW - H - 2), jnp.float32)], axis=1)
    vals = jnp.concatenate([h[:, :OUT] * p0, h[:, OUT:] * p1, tail], axis=1)
    vals_ref[0] = vals * msk


def _run_gru_attn(edata, wih_t, whh_t, bih_s, bhh_s, attn_s):
    grid = (2, E_PAD // B2)
    return pl.pallas_call(
        _gru_attn_body,
        grid=grid,
        in_specs=[
            pl.BlockSpec((1, L, B2, OUT), lambda m, j: (m, 0, j, 0)),
            pl.BlockSpec((1, OUT, 3 * H), lambda m, j: (m, 0, 0)),
            pl.BlockSpec((1, H, 3 * H), lambda m, j: (m, 0, 0)),
            pl.BlockSpec((1, 1, 3 * H), lambda m, j: (m, 0, 0)),
            pl.BlockSpec((1, 1, 3 * H), lambda m, j: (m, 0, 0)),
            pl.BlockSpec((1, 1, H), lambda m, j: (m, 0, 0)),
        ],
        out_specs=pl.BlockSpec((1, B2, VW), lambda m, j: (m, j, 0)),
        out_shape=jax.ShapeDtypeStruct((2, E_PAD, VW), jnp.float32),
    )(edata, wih_t, whh_t, bih_s, bhh_s, attn_s)


def _finalize_body(acc_ref, w1_ref, b1_ref, w2_ref, b2_ref, w3_ref, b3_ref,
                   m_ref, ssum_ref):
    j = pl.program_id(1)
    blk = acc_ref[0]
    den0 = blk[:, H:H + 1]
    den1 = blk[:, H + 1:H + 2]
    m0 = jnp.where(den0 > 0, blk[:, :OUT] / den0, 0.0)
    m1 = jnp.where(den1 > 0, blk[:, OUT:H] / den1, 0.0)
    m = jnp.concatenate([m0, m1], axis=1)
    m = jnp.where(m > 0, m, jnp.exp(jnp.minimum(m, 0.0)) - 1.0)
    m_ref[0] = m

    x = jax.nn.relu(jnp.dot(m, w1_ref[...], preferred_element_type=jnp.float32)
                    + b1_ref[...])
    x = jax.nn.relu(jnp.dot(x, w2_ref[...], preferred_element_type=jnp.float32)
                    + b2_ref[...])
    x = jax.nn.relu(jnp.dot(x, w3_ref[...], preferred_element_type=jnp.float32)
                    + b3_ref[...])
    x = jnp.tanh(x)
    part = jnp.sum(x, axis=0, keepdims=True)

    @pl.when(j == 0)
    def _():
        ssum_ref[0] = jnp.zeros_like(part)

    ssum_ref[0] += part


def _run_finalize(acc, w1t, b1, w2t, b2, w3t, b3):
    grid = (2, N // B4)
    return pl.pallas_call(
        _finalize_body,
        grid=grid,
        in_specs=[
            pl.BlockSpec((1, B4, VW), lambda m, j: (m, j, 0)),
            pl.BlockSpec((H, 2 * OUT), lambda m, j: (0, 0)),
            pl.BlockSpec((1, 2 * OUT), lambda m, j: (0, 0)),
            pl.BlockSpec((2 * OUT, OUT), lambda m, j: (0, 0)),
            pl.BlockSpec((1, OUT), lambda m, j: (0, 0)),
            pl.BlockSpec((OUT, AV), lambda m, j: (0, 0)),
            pl.BlockSpec((1, AV), lambda m, j: (0, 0)),
        ],
        out_specs=[
            pl.BlockSpec((1, B4, H), lambda m, j: (m, j, 0)),
            pl.BlockSpec((1, 1, AV), lambda m, j: (m, 0, 0)),
        ],
        out_shape=[
            jax.ShapeDtypeStruct((2, N, H), jnp.float32),
            jax.ShapeDtypeStruct((2, 1, AV), jnp.float32),
        ],
    )(acc, w1t, b1, w2t, b2, w3t, b3)


def _blend_body(m_ref, ssum_ref, fc2_ref, out_ref):
    s = ssum_ref[...]
    w = fc2_ref[...]
    l0 = jnp.sum(s[0] * w, keepdims=True) * (1.0 / N)
    l1 = jnp.sum(s[1] * w, keepdims=True) * (1.0 / N)
    b0 = jax.nn.sigmoid(l0 - l1)
    b1 = 1.0 - b0
    out_ref[...] = b0 * m_ref[0] + b1 * m_ref[1]


def _run_blend(m_all, ssum, fc2_w):
    grid = (N // B4,)
    return pl.pallas_call(
        _blend_body,
        grid=grid,
        in_specs=[
            pl.BlockSpec((2, B4, H), lambda j: (0, j, 0)),
            pl.BlockSpec((2, 1, AV), lambda j: (0, 0, 0)),
            pl.BlockSpec((1, AV), lambda j: (0, 0)),
        ],
        out_specs=pl.BlockSpec((B4, H), lambda j: (j, 0)),
        out_shape=jax.ShapeDtypeStruct((N, H), jnp.float32),
    )(m_all, ssum, fc2_w)


def kernel(features, edge_metapath_indices_0, edge_dst_0,
           edge_metapath_indices_1, edge_dst_1,
           gru_Wih_0, gru_Whh_0, gru_bih_0, gru_bhh_0, attn_0,
           gru_Wih_1, gru_Whh_1, gru_bih_1, gru_bhh_1, attn_1,
           fc1_w1, fc1_b1, fc1_w2, fc1_b2, fc1_w3, fc1_b3, fc2_w):
    f32 = jnp.float32

    def flat_idx(emi):
        emi_p = jnp.pad(emi, ((0, E_PAD - E), (0, 0)))
        return emi_p.astype(jnp.int32).T.reshape(-1)

    idx_all = jnp.concatenate(
        [flat_idx(edge_metapath_indices_0), flat_idx(edge_metapath_indices_1)]
    ).reshape(NW, G_CHUNKS, CHUNK)

    def dst_idx(dst):
        d = jnp.pad(dst.astype(jnp.int32), (0, E_PAD - E))
        return d.reshape(NS, S_CHUNKS, CHUNK)

    dst_all = jnp.stack([dst_idx(edge_dst_0), dst_idx(edge_dst_1)])

    edata = _gather_call(features.astype(f32), idx_all)
    edata = edata.reshape(2, L, E_PAD, OUT)

    wih_t = jnp.stack([gru_Wih_0.T, gru_Wih_1.T]).astype(f32)
    whh_t = jnp.stack([gru_Whh_0.T, gru_Whh_1.T]).astype(f32)
    bih_s = jnp.stack(
        [gru_bih_0, gru_bih_1]).reshape(2, 1, 3 * H).astype(f32)
    bhh_s = jnp.stack(
        [gru_bhh_0, gru_bhh_1]).reshape(2, 1, 3 * H).astype(f32)
    attn_s = jnp.stack(
        [attn_0.reshape(-1), attn_1.reshape(-1)]).reshape(2, 1, H).astype(f32)
    vals = _run_gru_attn(edata, wih_t, whh_t, bih_s, bhh_s, attn_s)

    zeros_hbm = jnp.zeros((N_PER_T, VW), f32)
    acc = _scatter_call(vals, dst_all, zeros_hbm)

    m_all, ssum = _run_finalize(
        acc, fc1_w1.T.astype(f32), fc1_b1.reshape(1, -1).astype(f32),
        fc1_w2.T.astype(f32), fc1_b2.reshape(1, -1).astype(f32),
        fc1_w3.T.astype(f32), fc1_b3.reshape(1, -1).astype(f32))

    return _run_blend(m_all, ssum, fc2_w.astype(f32))


# speedup vs baseline: 16.6546x; 1.0552x over previous
import functools

import jax
import jax.numpy as jnp
from jax import lax
from jax.experimental import pallas as pl
from jax.experimental.pallas import tpu as pltpu
from jax.experimental.pallas import tpu_sc as plsc

N = 10000
E = 160000
L = 3
OUT = 64
NH = 2
H = NH * OUT
AV = 128
VW = 144

NC = 2
NS = 16
NW = NC * NS

E_PAD = 163840
CHUNK = 128
G_PER_W = 2 * L * E_PAD // NW
G_CHUNKS = G_PER_W // CHUNK
S_PER_T = E_PAD // NS
S_CHUNKS = S_PER_T // CHUNK
N_PER_T = N // NS

B2 = 640
B4 = 1000


def _make_gather():
    mesh = plsc.VectorSubcoreMesh(core_axis_name="c", subcore_axis_name="s")

    @functools.partial(
        pl.kernel,
        mesh=mesh,
        out_type=jax.ShapeDtypeStruct((2 * L * E_PAD, OUT), jnp.float32),
        compiler_params=pltpu.CompilerParams(use_tc_tiling_on_sc=False),
        scratch_types=[
            pltpu.VMEM((G_CHUNKS, CHUNK), jnp.int32),
            pltpu.VMEM((2, CHUNK, OUT), jnp.float32),
            pltpu.SemaphoreType.DMA,
            pltpu.SemaphoreType.DMA,
        ],
    )
    def gather_k(feat_hbm, idx_hbm, out_hbm, idx_v, rows_v, sem0, sem1):
        c = lax.axis_index("c")
        s = lax.axis_index("s")
        wid = s * NC + c
        pltpu.sync_copy(idx_hbm.at[wid], idx_v)
        sems = (sem0, sem1)

        pltpu.async_copy(feat_hbm.at[idx_v.at[0]], rows_v.at[0], sem0)
        pltpu.async_copy(feat_hbm.at[idx_v.at[1]], rows_v.at[1], sem1)

        def body(j2, carry):
            for b in range(2):
                j = j2 * 2 + b
                buf = rows_v.at[b]
                pltpu.make_async_copy(
                    feat_hbm.at[idx_v.at[0]], buf, sems[b]).wait()
                pltpu.sync_copy(
                    buf, out_hbm.at[pl.ds(wid * G_PER_W + j * CHUNK, CHUNK)])

                @pl.when(j + 2 < G_CHUNKS)
                def _():
                    pltpu.async_copy(
                        feat_hbm.at[idx_v.at[j + 2]], buf, sems[b])
            return carry

        lax.fori_loop(0, G_CHUNKS // 2, body, 0)

    return gather_k


def _make_scatter():
    mesh = plsc.VectorSubcoreMesh(core_axis_name="c", subcore_axis_name="s")

    @functools.partial(
        pl.kernel,
        mesh=mesh,
        out_type=jax.ShapeDtypeStruct((2, N, VW), jnp.float32),
        compiler_params=pltpu.CompilerParams(use_tc_tiling_on_sc=False),
        scratch_types=[
            pltpu.VMEM((2, CHUNK), jnp.int32),
            pltpu.VMEM((2, CHUNK, VW), jnp.float32),
            pltpu.VMEM_SHARED((N, VW), jnp.float32),
            pltpu.SemaphoreType.DMA,
            pltpu.SemaphoreType.DMA,
            pltpu.SemaphoreType.DMA,
            pltpu.SemaphoreType.DMA,
        ],
    )
    def scatter_k(vals_hbm, dst_hbm, zeros_hbm, acc_hbm, idx_v, vbuf, shacc,
                  semv0, semv1, semi0, semi1):
        c = lax.axis_index("c")
        s = lax.axis_index("s")
        pltpu.sync_copy(zeros_hbm, shacc.at[pl.ds(s * N_PER_T, N_PER_T)])
        plsc.subcore_barrier()
        semv = (semv0, semv1)
        semi = (semi0, semi1)

        def vals_src(j):
            return vals_hbm.at[c, pl.ds(s * S_PER_T + j * CHUNK, CHUNK)]

        def idx_src(j):
            return dst_hbm.at[c, s, j]

        for b in range(2):
            pltpu.async_copy(vals_src(b), vbuf.at[b], semv[b])
            pltpu.async_copy(idx_src(b), idx_v.at[b], semi[b])

        def body(j2, carry):
            for b in range(2):
                j = j2 * 2 + b
                buf = vbuf.at[b]
                pltpu.make_async_copy(vals_src(0), buf, semv[b]).wait()
                pltpu.make_async_copy(
                    idx_src(0), idx_v.at[b], semi[b]).wait()
                pltpu.sync_copy(buf, shacc.at[idx_v.at[b]], add=True)

                @pl.when(j + 2 < S_CHUNKS)
                def _():
                    pltpu.async_copy(vals_src(j + 2), buf, semv[b])
                    pltpu.async_copy(idx_src(j + 2), idx_v.at[b], semi[b])
            return carry

        lax.fori_loop(0, S_CHUNKS // 2, body, 0)
        plsc.subcore_barrier()
        pltpu.sync_copy(
            shacc.at[pl.ds(s * N_PER_T, N_PER_T)],
            acc_hbm.at[c, pl.ds(s * N_PER_T, N_PER_T)])

    return scatter_k


_gather_call = _make_gather()
_scatter_call = _make_scatter()


def _gru_attn_body(ed_ref, wih_ref, whh_ref, bih_ref, bhh_ref, attn_ref,
                   vals_ref):
    j = pl.program_id(1)
    wih = wih_ref[0]
    whh = whh_ref[0]
    bih = bih_ref[0]
    bhh = bhh_ref[0]
    att = attn_ref[0]

    gis = [
        jnp.dot(ed_ref[0, l], wih, preferred_element_type=jnp.float32) + bih
        for l in range(L)
    ]

    h = None
    for l in range(L):
        gi = gis[l]
        if h is None:
            gh = jnp.broadcast_to(bhh, gi.shape)
        else:
            gh = jnp.dot(h, whh, preferred_element_type=jnp.float32) + bhh
        r = jax.nn.sigmoid(gi[:, :H] + gh[:, :H])
        z = jax.nn.sigmoid(gi[:, H:2 * H] + gh[:, H:2 * H])
        n = jnp.tanh(gi[:, 2 * H:] + r * gh[:, 2 * H:])
        h = (1.0 - z) * n if l == 0 else (1.0 - z) * n + z * h

    a0 = jnp.sum(h[:, :OUT] * att[:, :OUT], axis=-1, keepdims=True)
    a1 = jnp.sum(h[:, OUT:] * att[:, OUT:], axis=-1, keepdims=True)
    a0 = jnp.where(a0 >= 0, a0, 0.01 * a0)
    a1 = jnp.where(a1 >= 0, a1, 0.01 * a1)
    p0 = jnp.exp(a0)
    p1 = jnp.exp(a1)

    e0 = j * B2 + lax.broadcasted_iota(jnp.int32, (B2, 1), 0)
    msk = (e0 < E).astype(jnp.float32)
    tail = jnp.concatenate(
        [p0, p1, jnp.zeros((B2, VW - H - 2), jnp.float32)], axis=1)
    vals = jnp.concatenate([h[:, :OUT] * p0, h[:, OUT:] * p1, tail], axis=1)
    vals_ref[0] = vals * msk


def _run_gru_attn(edata, wih_t, whh_t, bih_s, bhh_s, attn_s):
    grid = (2, E_PAD // B2)
    return pl.pallas_call(
        _gru_attn_body,
        grid=grid,
        in_specs=[
            pl.BlockSpec((1, L, B2, OUT), lambda m, j: (m, 0, j, 0)),
            pl.BlockSpec((1, OUT, 3 * H), lambda m, j: (m, 0, 0)),
            pl.BlockSpec((1, H, 3 * H), lambda m, j: (m, 0, 0)),
            pl.BlockSpec((1, 1, 3 * H), lambda m, j: (m, 0, 0)),
            pl.BlockSpec((1, 1, 3 * H), lambda m, j: (m, 0, 0)),
            pl.BlockSpec((1, 1, H), lambda m, j: (m, 0, 0)),
        ],
        out_specs=pl.BlockSpec((1, B2, VW), lambda m, j: (m, j, 0)),
        out_shape=jax.ShapeDtypeStruct((2, E_PAD, VW), jnp.float32),
    )(edata, wih_t, whh_t, bih_s, bhh_s, attn_s)


def _finalize_body(acc_ref, w1_ref, b1_ref, w2_ref, b2_ref, w3_ref, b3_ref,
                   m_ref, ssum_ref):
    j = pl.program_id(1)
    blk = acc_ref[0]
    den0 = blk[:, H:H + 1]
    den1 = blk[:, H + 1:H + 2]
    m0 = jnp.where(den0 > 0, blk[:, :OUT] / den0, 0.0)
    m1 = jnp.where(den1 > 0, blk[:, OUT:H] / den1, 0.0)
    m = jnp.concatenate([m0, m1], axis=1)
    m = jnp.where(m > 0, m, jnp.exp(jnp.minimum(m, 0.0)) - 1.0)
    m_ref[0] = m

    x = jax.nn.relu(jnp.dot(m, w1_ref[...], preferred_element_type=jnp.float32)
                    + b1_ref[...])
    x = jax.nn.relu(jnp.dot(x, w2_ref[...], preferred_element_type=jnp.float32)
                    + b2_ref[...])
    x = jax.nn.relu(jnp.dot(x, w3_ref[...], preferred_element_type=jnp.float32)
                    + b3_ref[...])
    x = jnp.tanh(x)
    part = jnp.sum(x, axis=0, keepdims=True)

    @pl.when(j == 0)
    def _():
        ssum_ref[0] = jnp.zeros_like(part)

    ssum_ref[0] += part


def _run_finalize(acc, w1t, b1, w2t, b2, w3t, b3):
    grid = (2, N // B4)
    return pl.pallas_call(
        _finalize_body,
        grid=grid,
        in_specs=[
            pl.BlockSpec((1, B4, VW), lambda m, j: (m, j, 0)),
            pl.BlockSpec((H, 2 * OUT), lambda m, j: (0, 0)),
            pl.BlockSpec((1, 2 * OUT), lambda m, j: (0, 0)),
            pl.BlockSpec((2 * OUT, OUT), lambda m, j: (0, 0)),
            pl.BlockSpec((1, OUT), lambda m, j: (0, 0)),
            pl.BlockSpec((OUT, AV), lambda m, j: (0, 0)),
            pl.BlockSpec((1, AV), lambda m, j: (0, 0)),
        ],
        out_specs=[
            pl.BlockSpec((1, B4, H), lambda m, j: (m, j, 0)),
            pl.BlockSpec((1, 1, AV), lambda m, j: (m, 0, 0)),
        ],
        out_shape=[
            jax.ShapeDtypeStruct((2, N, H), jnp.float32),
            jax.ShapeDtypeStruct((2, 1, AV), jnp.float32),
        ],
    )(acc, w1t, b1, w2t, b2, w3t, b3)


def _blend_body(m_ref, ssum_ref, fc2_ref, out_ref):
    s = ssum_ref[...]
    w = fc2_ref[...]
    l0 = jnp.sum(s[0] * w, keepdims=True) * (1.0 / N)
    l1 = jnp.sum(s[1] * w, keepdims=True) * (1.0 / N)
    b0 = jax.nn.sigmoid(l0 - l1)
    b1 = 1.0 - b0
    out_ref[...] = b0 * m_ref[0] + b1 * m_ref[1]


def _run_blend(m_all, ssum, fc2_w):
    grid = (N // B4,)
    return pl.pallas_call(
        _blend_body,
        grid=grid,
        in_specs=[
            pl.BlockSpec((2, B4, H), lambda j: (0, j, 0)),
            pl.BlockSpec((2, 1, AV), lambda j: (0, 0, 0)),
            pl.BlockSpec((1, AV), lambda j: (0, 0)),
        ],
        out_specs=pl.BlockSpec((B4, H), lambda j: (j, 0)),
        out_shape=jax.ShapeDtypeStruct((N, H), jnp.float32),
    )(m_all, ssum, fc2_w)


def kernel(features, edge_metapath_indices_0, edge_dst_0,
           edge_metapath_indices_1, edge_dst_1,
           gru_Wih_0, gru_Whh_0, gru_bih_0, gru_bhh_0, attn_0,
           gru_Wih_1, gru_Whh_1, gru_bih_1, gru_bhh_1, attn_1,
           fc1_w1, fc1_b1, fc1_w2, fc1_b2, fc1_w3, fc1_b3, fc2_w):
    f32 = jnp.float32

    def flat_idx(emi):
        emi_p = jnp.pad(emi, ((0, E_PAD - E), (0, 0)))
        return emi_p.astype(jnp.int32).T.reshape(-1)

    idx_all = jnp.concatenate(
        [flat_idx(edge_metapath_indices_0), flat_idx(edge_metapath_indices_1)]
    ).reshape(NW, G_CHUNKS, CHUNK)

    def dst_idx(dst):
        d = jnp.pad(dst.astype(jnp.int32), (0, E_PAD - E))
        return d.reshape(NS, S_CHUNKS, CHUNK)

    dst_all = jnp.stack([dst_idx(edge_dst_0), dst_idx(edge_dst_1)])

    edata = _gather_call(features.astype(f32), idx_all)
    edata = edata.reshape(2, L, E_PAD, OUT)

    wih_t = jnp.stack([gru_Wih_0.T, gru_Wih_1.T]).astype(f32)
    whh_t = jnp.stack([gru_Whh_0.T, gru_Whh_1.T]).astype(f32)
    bih_s = jnp.stack(
        [gru_bih_0, gru_bih_1]).reshape(2, 1, 3 * H).astype(f32)
    bhh_s = jnp.stack(
        [gru_bhh_0, gru_bhh_1]).reshape(2, 1, 3 * H).astype(f32)
    attn_s = jnp.stack(
        [attn_0.reshape(-1), attn_1.reshape(-1)]).reshape(2, 1, H).astype(f32)
    vals = _run_gru_attn(edata, wih_t, whh_t, bih_s, bhh_s, attn_s)

    zeros_hbm = jnp.zeros((N_PER_T, VW), f32)
    acc = _scatter_call(vals, dst_all, zeros_hbm)

    m_all, ssum = _run_finalize(
        acc, fc1_w1.T.astype(f32), fc1_b1.reshape(1, -1).astype(f32),
        fc1_w2.T.astype(f32), fc1_b2.reshape(1, -1).astype(f32),
        fc1_w3.T.astype(f32), fc1_b3.reshape(1, -1).astype(f32))

    return _run_blend(m_all, ssum, fc2_w.astype(f32))


# speedup vs baseline: 16.9848x; 1.0198x over previous
import functools

import jax
import jax.numpy as jnp
from jax import lax
from jax.experimental import pallas as pl
from jax.experimental.pallas import tpu as pltpu
from jax.experimental.pallas import tpu_sc as plsc

N = 10000
E = 160000
L = 3
OUT = 64
NH = 2
H = NH * OUT
AV = 128
VW = 144

NC = 2
NS = 16
NW = NC * NS

E_PAD = 163840
CHUNK = 128
G_PER_W = 2 * L * E_PAD // NW
G_CHUNKS = G_PER_W // CHUNK
S_PER_T = E_PAD // NS
S_CHUNKS = S_PER_T // CHUNK
N_PER_T = N // NS

B2 = 640
B4 = 1000


def _make_gather():
    mesh = plsc.VectorSubcoreMesh(core_axis_name="c", subcore_axis_name="s")

    @functools.partial(
        pl.kernel,
        mesh=mesh,
        out_type=jax.ShapeDtypeStruct((2 * L * E_PAD, OUT), jnp.float32),
        compiler_params=pltpu.CompilerParams(use_tc_tiling_on_sc=False),
        scratch_types=[
            pltpu.VMEM((G_CHUNKS, CHUNK), jnp.int32),
            pltpu.VMEM((2, CHUNK, OUT), jnp.float32),
            pltpu.SemaphoreType.DMA,
            pltpu.SemaphoreType.DMA,
        ],
    )
    def gather_k(feat_hbm, idx_hbm, out_hbm, idx_v, rows_v, sem0, sem1):
        c = lax.axis_index("c")
        s = lax.axis_index("s")
        wid = s * NC + c
        pltpu.sync_copy(idx_hbm.at[wid], idx_v)
        sems = (sem0, sem1)

        pltpu.async_copy(feat_hbm.at[idx_v.at[0]], rows_v.at[0], sem0)
        pltpu.async_copy(feat_hbm.at[idx_v.at[1]], rows_v.at[1], sem1)

        def body(j2, carry):
            for b in range(2):
                j = j2 * 2 + b
                buf = rows_v.at[b]
                pltpu.make_async_copy(
                    feat_hbm.at[idx_v.at[0]], buf, sems[b]).wait()
                pltpu.sync_copy(
                    buf, out_hbm.at[pl.ds(wid * G_PER_W + j * CHUNK, CHUNK)])

                @pl.when(j + 2 < G_CHUNKS)
                def _():
                    pltpu.async_copy(
                        feat_hbm.at[idx_v.at[j + 2]], buf, sems[b])
            return carry

        lax.fori_loop(0, G_CHUNKS // 2, body, 0)

    return gather_k


def _make_scatter():
    mesh = plsc.VectorSubcoreMesh(core_axis_name="c", subcore_axis_name="s")

    @functools.partial(
        pl.kernel,
        mesh=mesh,
        out_type=jax.ShapeDtypeStruct((2, N, VW), jnp.float32),
        compiler_params=pltpu.CompilerParams(use_tc_tiling_on_sc=False),
        scratch_types=[
            pltpu.VMEM((2, CHUNK), jnp.int32),
            pltpu.VMEM((2, CHUNK, VW), jnp.float32),
            pltpu.VMEM_SHARED((N, VW), jnp.float32),
            pltpu.SemaphoreType.DMA,
            pltpu.SemaphoreType.DMA,
            pltpu.SemaphoreType.DMA,
            pltpu.SemaphoreType.DMA,
        ],
    )
    def scatter_k(vals_hbm, dst_hbm, zeros_hbm, acc_hbm, idx_v, vbuf, shacc,
                  semv0, semv1, semi0, semi1):
        c = lax.axis_index("c")
        s = lax.axis_index("s")
        pltpu.sync_copy(zeros_hbm, shacc.at[pl.ds(s * N_PER_T, N_PER_T)])
        plsc.subcore_barrier()
        semv = (semv0, semv1)
        semi = (semi0, semi1)

        def vals_src(j):
            return vals_hbm.at[c, pl.ds(s * S_PER_T + j * CHUNK, CHUNK)]

        def idx_src(j):
            return dst_hbm.at[c, s, j]

        for b in range(2):
            pltpu.async_copy(vals_src(b), vbuf.at[b], semv[b])
            pltpu.async_copy(idx_src(b), idx_v.at[b], semi[b])

        def body(j2, carry):
            for b in range(2):
                j = j2 * 2 + b
                buf = vbuf.at[b]
                pltpu.make_async_copy(vals_src(0), buf, semv[b]).wait()
                pltpu.make_async_copy(
                    idx_src(0), idx_v.at[b], semi[b]).wait()
                pltpu.sync_copy(buf, shacc.at[idx_v.at[b]], add=True)

                @pl.when(j + 2 < S_CHUNKS)
                def _():
                    pltpu.async_copy(vals_src(j + 2), buf, semv[b])
                    pltpu.async_copy(idx_src(j + 2), idx_v.at[b], semi[b])
            return carry

        lax.fori_loop(0, S_CHUNKS // 2, body, 0)
        plsc.subcore_barrier()
        pltpu.sync_copy(
            shacc.at[pl.ds(s * N_PER_T, N_PER_T)],
            acc_hbm.at[c, pl.ds(s * N_PER_T, N_PER_T)])

    return scatter_k


_gather_call = _make_gather()
_scatter_call = _make_scatter()


def _gru_attn_body(ed_ref, wrz_ref, win_ref, whn_ref, bih_ref, bhh_ref,
                   attn_ref, vals_ref):
    j = pl.program_id(1)
    wrz = wrz_ref[0]
    win = win_ref[0]
    whn = whn_ref[0]
    bih = bih_ref[0]
    bhh = bhh_ref[0]
    att = attn_ref[0]
    bsum_rz = bih[:, :2 * H] + bhh[:, :2 * H]
    bih_n = bih[:, 2 * H:]
    bhh_n = bhh[:, 2 * H:]

    xs = [ed_ref[0, l].astype(jnp.bfloat16) for l in range(L)]

    h = None
    for l in range(L):
        gin = jnp.dot(xs[l], win, preferred_element_type=jnp.float32) + bih_n
        if h is None:
            rz = (jnp.dot(xs[l], wrz[:OUT],
                          preferred_element_type=jnp.float32) + bsum_rz)
            ghn = bhh_n
        else:
            hb = h.astype(jnp.bfloat16)
            xh = jnp.concatenate([xs[l], hb], axis=1)
            rz = (jnp.dot(xh, wrz, preferred_element_type=jnp.float32)
                  + bsum_rz)
            ghn = (jnp.dot(hb, whn, preferred_element_type=jnp.float32)
                   + bhh_n)
        r = jax.nn.sigmoid(rz[:, :H])
        z = jax.nn.sigmoid(rz[:, H:])
        n = jnp.tanh(gin + r * ghn)
        h = (1.0 - z) * n if l == 0 else (1.0 - z) * n + z * h

    a0 = jnp.sum(h[:, :OUT] * att[:, :OUT], axis=-1, keepdims=True)
    a1 = jnp.sum(h[:, OUT:] * att[:, OUT:], axis=-1, keepdims=True)
    a0 = jnp.where(a0 >= 0, a0, 0.01 * a0)
    a1 = jnp.where(a1 >= 0, a1, 0.01 * a1)
    p0 = jnp.exp(a0)
    p1 = jnp.exp(a1)

    e0 = j * B2 + lax.broadcasted_iota(jnp.int32, (B2, 1), 0)
    msk = (e0 < E).astype(jnp.float32)
    tail = jnp.concatenate(
        [p0, p1, jnp.zeros((B2, VW - H - 2), jnp.float32)], axis=1)
    vals = jnp.concatenate([h[:, :OUT] * p0, h[:, OUT:] * p1, tail], axis=1)
    vals_ref[0] = vals * msk


def _run_gru_attn(edata, wrz_s, win_s, whn_s, bih_s, bhh_s, attn_s):
    grid = (2, E_PAD // B2)
    return pl.pallas_call(
        _gru_attn_body,
        grid=grid,
        in_specs=[
            pl.BlockSpec((1, L, B2, OUT), lambda m, j: (m, 0, j, 0)),
            pl.BlockSpec((1, OUT + H, 2 * H), lambda m, j: (m, 0, 0)),
            pl.BlockSpec((1, OUT, H), lambda m, j: (m, 0, 0)),
            pl.BlockSpec((1, H, H), lambda m, j: (m, 0, 0)),
            pl.BlockSpec((1, 1, 3 * H), lambda m, j: (m, 0, 0)),
            pl.BlockSpec((1, 1, 3 * H), lambda m, j: (m, 0, 0)),
            pl.BlockSpec((1, 1, H), lambda m, j: (m, 0, 0)),
        ],
        out_specs=pl.BlockSpec((1, B2, VW), lambda m, j: (m, j, 0)),
        out_shape=jax.ShapeDtypeStruct((2, E_PAD, VW), jnp.float32),
    )(edata, wrz_s, win_s, whn_s, bih_s, bhh_s, attn_s)


def _finalize_body(acc_ref, w1_ref, b1_ref, w2_ref, b2_ref, w3_ref, b3_ref,
                   m_ref, ssum_ref):
    j = pl.program_id(1)
    blk = acc_ref[0]
    den0 = blk[:, H:H + 1]
    den1 = blk[:, H + 1:H + 2]
    m0 = jnp.where(den0 > 0, blk[:, :OUT] / den0, 0.0)
    m1 = jnp.where(den1 > 0, blk[:, OUT:H] / den1, 0.0)
    m = jnp.concatenate([m0, m1], axis=1)
    m = jnp.where(m > 0, m, jnp.exp(jnp.minimum(m, 0.0)) - 1.0)
    m_ref[0] = m

    x = jax.nn.relu(jnp.dot(m, w1_ref[...], preferred_element_type=jnp.float32)
                    + b1_ref[...])
    x = jax.nn.relu(jnp.dot(x, w2_ref[...], preferred_element_type=jnp.float32)
                    + b2_ref[...])
    x = jax.nn.relu(jnp.dot(x, w3_ref[...], preferred_element_type=jnp.float32)
                    + b3_ref[...])
    x = jnp.tanh(x)
    part = jnp.sum(x, axis=0, keepdims=True)

    @pl.when(j == 0)
    def _():
        ssum_ref[0] = jnp.zeros_like(part)

    ssum_ref[0] += part


def _run_finalize(acc, w1t, b1, w2t, b2, w3t, b3):
    grid = (2, N // B4)
    return pl.pallas_call(
        _finalize_body,
        grid=grid,
        in_specs=[
            pl.BlockSpec((1, B4, VW), lambda m, j: (m, j, 0)),
            pl.BlockSpec((H, 2 * OUT), lambda m, j: (0, 0)),
            pl.BlockSpec((1, 2 * OUT), lambda m, j: (0, 0)),
            pl.BlockSpec((2 * OUT, OUT), lambda m, j: (0, 0)),
            pl.BlockSpec((1, OUT), lambda m, j: (0, 0)),
            pl.BlockSpec((OUT, AV), lambda m, j: (0, 0)),
            pl.BlockSpec((1, AV), lambda m, j: (0, 0)),
        ],
        out_specs=[
            pl.BlockSpec((1, B4, H), lambda m, j: (m, j, 0)),
            pl.BlockSpec((1, 1, AV), lambda m, j: (m, 0, 0)),
        ],
        out_shape=[
            jax.ShapeDtypeStruct((2, N, H), jnp.float32),
            jax.ShapeDtypeStruct((2, 1, AV), jnp.float32),
        ],
    )(acc, w1t, b1, w2t, b2, w3t, b3)


def _blend_body(m_ref, ssum_ref, fc2_ref, out_ref):
    s = ssum_ref[...]
    w = fc2_ref[...]
    l0 = jnp.sum(s[0] * w, keepdims=True) * (1.0 / N)
    l1 = jnp.sum(s[1] * w, keepdims=True) * (1.0 / N)
    b0 = jax.nn.sigmoid(l0 - l1)
    b1 = 1.0 - b0
    out_ref[...] = b0 * m_ref[0] + b1 * m_ref[1]


def _run_blend(m_all, ssum, fc2_w):
    grid = (N // B4,)
    return pl.pallas_call(
        _blend_body,
        grid=grid,
        in_specs=[
            pl.BlockSpec((2, B4, H), lambda j: (0, j, 0)),
            pl.BlockSpec((2, 1, AV), lambda j: (0, 0, 0)),
            pl.BlockSpec((1, AV), lambda j: (0, 0)),
        ],
        out_specs=pl.BlockSpec((B4, H), lambda j: (j, 0)),
        out_shape=jax.ShapeDtypeStruct((N, H), jnp.float32),
    )(m_all, ssum, fc2_w)


def kernel(features, edge_metapath_indices_0, edge_dst_0,
           edge_metapath_indices_1, edge_dst_1,
           gru_Wih_0, gru_Whh_0, gru_bih_0, gru_bhh_0, attn_0,
           gru_Wih_1, gru_Whh_1, gru_bih_1, gru_bhh_1, attn_1,
           fc1_w1, fc1_b1, fc1_w2, fc1_b2, fc1_w3, fc1_b3, fc2_w):
    f32 = jnp.float32

    def flat_idx(emi):
        emi_p = jnp.pad(emi, ((0, E_PAD - E), (0, 0)))
        return emi_p.astype(jnp.int32).T.reshape(-1)

    idx_all = jnp.concatenate(
        [flat_idx(edge_metapath_indices_0), flat_idx(edge_metapath_indices_1)]
    ).reshape(NW, G_CHUNKS, CHUNK)

    def dst_idx(dst):
        d = jnp.pad(dst.astype(jnp.int32), (0, E_PAD - E))
        return d.reshape(NS, S_CHUNKS, CHUNK)

    dst_all = jnp.stack([dst_idx(edge_dst_0), dst_idx(edge_dst_1)])

    edata = _gather_call(features.astype(f32), idx_all)
    edata = edata.reshape(2, L, E_PAD, OUT)

    bf16 = jnp.bfloat16

    def grurz(wih, whh):
        return jnp.concatenate(
            [wih.T[:, :2 * H], whh.T[:, :2 * H]], axis=0).astype(bf16)

    wrz_s = jnp.stack([grurz(gru_Wih_0, gru_Whh_0),
                       grurz(gru_Wih_1, gru_Whh_1)])
    win_s = jnp.stack([gru_Wih_0.T[:, 2 * H:],
                       gru_Wih_1.T[:, 2 * H:]]).astype(bf16)
    whn_s = jnp.stack([gru_Whh_0.T[:, 2 * H:],
                       gru_Whh_1.T[:, 2 * H:]]).astype(bf16)
    bih_s = jnp.stack(
        [gru_bih_0, gru_bih_1]).reshape(2, 1, 3 * H).astype(f32)
    bhh_s = jnp.stack(
        [gru_bhh_0, gru_bhh_1]).reshape(2, 1, 3 * H).astype(f32)
    attn_s = jnp.stack(
        [attn_0.reshape(-1), attn_1.reshape(-1)]).reshape(2, 1, H).astype(f32)
    vals = _run_gru_attn(edata, wrz_s, win_s, whn_s, bih_s, bhh_s, attn_s)

    zeros_hbm = jnp.zeros((N_PER_T, VW), f32)
    acc = _scatter_call(vals, dst_all, zeros_hbm)

    m_all, ssum = _run_finalize(
        acc, fc1_w1.T.astype(f32), fc1_b1.reshape(1, -1).astype(f32),
        fc1_w2.T.astype(f32), fc1_b2.reshape(1, -1).astype(f32),
        fc1_w3.T.astype(f32), fc1_b3.reshape(1, -1).astype(f32))

    return _run_blend(m_all, ssum, fc2_w.astype(f32))
